# baseline (speedup 1.0000x reference)
import functools

import jax
import jax.numpy as jnp
from jax import lax
from jax.experimental import pallas as pl
from jax.experimental.pallas import tpu as pltpu
from jax.experimental.pallas import tpu_sc as plsc

N = 10000
E = 320000
NPAD = 10240
ROWS_PT = NPAD // 16
NACC = 10112
ROWS_ACC = NACC // 16
CH = 56
KG = 6
SUPER = 16
GROUPS = (6, 5, 5)
CPT = 368
EP = 16 * CPT * CH
CPT4 = 192
EP4 = 32 * CPT4 * CH
NSUP = CPT // SUPER
NSUP4 = CPT4 // SUPER

TN = 1024
GRID = NPAD // TN

f32 = jnp.float32
i32 = jnp.int32

_MESH = dict(core_axis_name="c", subcore_axis_name="s")


def _seg_loop(tab, src_h, dst_h, w, nsup, src_v, dst_v, bufs, acc, sem, sem2,
              gather=True):
  def sup(u, carry):
    if gather:
      pltpu.sync_copy(src_h.at[w, pl.ds(u * (SUPER * CH), SUPER * CH)],
                      src_v)
    pltpu.sync_copy(dst_h.at[w, pl.ds(u * SUPER, SUPER)], dst_v)

    base = 0
    for gs in GROUPS:
      if gather:
        gd = [pltpu.async_copy(
            tab.at[src_v.at[pl.ds((base + k) * CH, CH)]], bufs[k], sem)
            for k in range(gs)]
      sd = []
      for k in range(gs):
        if gather:
          gd[k].wait()
        sd.append(pltpu.async_copy(bufs[k], acc.at[dst_v.at[base + k]],
                                   sem2, add=True))
      for d in sd:
        d.wait()
      base += gs
    return carry
  lax.fori_loop(0, nsup, sup, 0)


def _make_sc_layer1():
  out_type = [jax.ShapeDtypeStruct((NPAD, 128), f32),
              jax.ShapeDtypeStruct((NPAD, 128), f32)]
  scratch = (
      [pltpu.VMEM((SUPER * CH,), i32), pltpu.VMEM((SUPER, CH), i32)]
      + [pltpu.VMEM((CH, 128), f32) for _ in range(KG)]
      + [pltpu.VMEM_SHARED((NACC, 128), f32),
         pltpu.SemaphoreType.DMA, pltpu.SemaphoreType.DMA])

  def body(src_h, dst_h, xtab, zrows, e0_h, agg_out, cnt_out,
           src_v, dst_v, *rest):
    bufs = list(rest[:KG])
    acc, sem, sem2 = rest[KG:]
    c = lax.axis_index("c")
    s = lax.axis_index("s")
    rows = pl.ds(s * ROWS_ACC, ROWS_ACC)

    pltpu.sync_copy(zrows, acc.at[rows])

    @pl.when(c == 1)
    def _():
      for b in bufs:
        pltpu.sync_copy(e0_h, b)

    plsc.subcore_barrier()

    @pl.when(c == 0)
    def _():
      _seg_loop(xtab, src_h, dst_h, s, NSUP, src_v, dst_v, bufs, acc,
                sem, sem2)

    @pl.when(c == 1)
    def _():
      _seg_loop(xtab, src_h, dst_h, s, NSUP, src_v, dst_v, bufs, acc,
                sem, sem2, gather=False)

    plsc.subcore_barrier()

    @pl.when(c == 0)
    def _():
      pltpu.sync_copy(acc.at[rows], agg_out.at[rows])

    @pl.when(c == 1)
    def _():
      pltpu.sync_copy(acc.at[rows], cnt_out.at[rows])

  return pl.kernel(body, out_type=out_type,
                   mesh=plsc.VectorSubcoreMesh(**_MESH),
                   scratch_types=scratch)


def _make_sc_mid():
  out_type = [jax.ShapeDtypeStruct((NPAD, 128), f32),
              jax.ShapeDtypeStruct((NPAD, 128), f32)]
  scratch = (
      [pltpu.VMEM((SUPER * CH,), i32), pltpu.VMEM((SUPER, CH), i32)]
      + [pltpu.VMEM((CH, 128), f32) for _ in range(KG)]
      + [pltpu.VMEM_SHARED((NACC, 128), f32),
         pltpu.SemaphoreType.DMA, pltpu.SemaphoreType.DMA])

  def body(src_h, dst_h, tabL, tabR, zrows, outL, outR, src_v, dst_v, *rest):
    bufs = list(rest[:KG])
    acc, sem, sem2 = rest[KG:]
    c = lax.axis_index("c")
    s = lax.axis_index("s")
    rows = pl.ds(s * ROWS_ACC, ROWS_ACC)

    pltpu.sync_copy(zrows, acc.at[rows])
    plsc.subcore_barrier()

    @pl.when(c == 0)
    def _():
      _seg_loop(tabL, src_h, dst_h, s, NSUP, src_v, dst_v, bufs, acc,
                sem, sem2)

    @pl.when(c == 1)
    def _():
      _seg_loop(tabR, src_h, dst_h, s, NSUP, src_v, dst_v, bufs, acc,
                sem, sem2)

    plsc.subcore_barrier()

    @pl.when(c == 0)
    def _():
      pltpu.sync_copy(acc.at[rows], outL.at[rows])

    @pl.when(c == 1)
    def _():
      pltpu.sync_copy(acc.at[rows], outR.at[rows])

  return pl.kernel(body, out_type=out_type,
                   mesh=plsc.VectorSubcoreMesh(**_MESH),
                   scratch_types=scratch)


def _make_sc_l4():
  out_type = [jax.ShapeDtypeStruct((NPAD, 128), f32),
              jax.ShapeDtypeStruct((NPAD, 128), f32)]
  scratch = (
      [pltpu.VMEM((SUPER * CH,), i32), pltpu.VMEM((SUPER, CH), i32)]
      + [pltpu.VMEM((CH, 128), f32) for _ in range(KG)]
      + [pltpu.VMEM_SHARED((NACC, 128), f32),
         pltpu.SemaphoreType.DMA, pltpu.SemaphoreType.DMA])

  def body(src_h, dst_h, ytab, zrows, pA, pB, src_v, dst_v, *rest):
    bufs = list(rest[:KG])
    acc, sem, sem2 = rest[KG:]
    c = lax.axis_index("c")
    s = lax.axis_index("s")
    w = c * 16 + s
    rows = pl.ds(s * ROWS_ACC, ROWS_ACC)

    pltpu.sync_copy(zrows, acc.at[rows])
    plsc.subcore_barrier()

    _seg_loop(ytab, src_h, dst_h, w, NSUP4, src_v, dst_v, bufs, acc,
              sem, sem2)

    plsc.subcore_barrier()

    @pl.when(c == 0)
    def _():
      pltpu.sync_copy(acc.at[rows], pA.at[rows])

    @pl.when(c == 1)
    def _():
      pltpu.sync_copy(acc.at[rows], pB.at[rows])

  return pl.kernel(body, out_type=out_type,
                   mesh=plsc.VectorSubcoreMesh(**_MESH),
                   scratch_types=scratch)


def _row_spec(w):
  return pl.BlockSpec((TN, w), lambda i: (i, 0))


def _w_spec(r, cols):
  return pl.BlockSpec((r, cols), lambda i: (0, 0))


def _make_tc1():
  def body(x_ref, agg, cnt, wl, bl, wr, g, bln, wres, brs, outL, outR):
    scale = 1.0 / jnp.maximum(cnt[...][:, :1], 1.0)
    seg = jnp.dot(agg[...], wl[...], preferred_element_type=f32)
    pre = (seg * scale + bl[...]
           + jnp.dot(x_ref[...], wr[...], preferred_element_type=f32))
    mu = jnp.mean(pre, axis=1, keepdims=True)
    var = jnp.mean((pre - mu) ** 2, axis=1, keepdims=True)
    h = (pre - mu) * lax.rsqrt(var + 1e-5) * g[...] + bln[...]
    h = (jnp.maximum(h, 0.0)
         + jnp.dot(x_ref[...], wres[...], preferred_element_type=f32)
         + brs[...])
    outL[...] = h[:, :128]
    outR[...] = h[:, 128:]

  return pl.pallas_call(
      body,
      grid=(GRID,),
      in_specs=[
          _row_spec(128), _row_spec(128), _row_spec(128),
          _w_spec(128, 256), _w_spec(1, 256), _w_spec(128, 256),
          _w_spec(1, 256), _w_spec(1, 256),
          _w_spec(128, 256), _w_spec(1, 256),
      ],
      out_specs=[_row_spec(128), _row_spec(128)],
      out_shape=[jax.ShapeDtypeStruct((NPAD, 128), f32)] * 2,
  )


def _make_tc_mid():
  def body(hL, hR, aL, aR, cnt, wlt, wlb, bl, wrt, wrb, outL, outR):
    scale = 1.0 / jnp.maximum(cnt[...][:, :1], 1.0)
    seg = (jnp.dot(aL[...], wlt[...], preferred_element_type=f32)
           + jnp.dot(aR[...], wlb[...], preferred_element_type=f32))
    h = (seg * scale + bl[...]
         + jnp.dot(hL[...], wrt[...], preferred_element_type=f32)
         + jnp.dot(hR[...], wrb[...], preferred_element_type=f32))
    h = jnp.maximum(h, 0.0)
    outL[...] = h[:, :128]
    outR[...] = h[:, 128:]

  return pl.pallas_call(
      body,
      grid=(GRID,),
      in_specs=[
          _row_spec(128), _row_spec(128), _row_spec(128), _row_spec(128),
          _row_spec(128),
          _w_spec(128, 256), _w_spec(128, 256), _w_spec(1, 256),
          _w_spec(128, 256), _w_spec(128, 256),
      ],
      out_specs=[_row_spec(128), _row_spec(128)],
      out_shape=[jax.ShapeDtypeStruct((NPAD, 128), f32)] * 2,
  )


def _make_tc3():
  def body(hL, hR, aL, aR, cnt, wlt, wlb, bl, wrt, wrb, w4l, w4r, b4,
           y_out, r_out):
    scale = 1.0 / jnp.maximum(cnt[...][:, :1], 1.0)
    seg = (jnp.dot(aL[...], wlt[...], preferred_element_type=f32)
           + jnp.dot(aR[...], wlb[...], preferred_element_type=f32))
    h = (seg * scale + bl[...]
         + jnp.dot(hL[...], wrt[...], preferred_element_type=f32)
         + jnp.dot(hR[...], wrb[...], preferred_element_type=f32))
    h = jnp.maximum(h, 0.0)
    y_out[...] = jnp.dot(h, w4l[...], preferred_element_type=f32)
    r_out[...] = jnp.dot(h, w4r[...], preferred_element_type=f32) + b4[...]

  return pl.pallas_call(
      body,
      grid=(GRID,),
      in_specs=[
          _row_spec(128), _row_spec(128), _row_spec(128), _row_spec(128),
          _row_spec(128),
          _w_spec(128, 256), _w_spec(128, 256), _w_spec(1, 256),
          _w_spec(128, 256), _w_spec(128, 256),
          _w_spec(256, 128), _w_spec(256, 128), _w_spec(1, 128),
      ],
      out_specs=[_row_spec(128), _row_spec(128)],
      out_shape=[jax.ShapeDtypeStruct((NPAD, 128), f32)] * 2,
  )


def _make_tc4():
  def body(pA, pB, cnt, r, out):
    scale = 1.0 / jnp.maximum(cnt[...][:, :1], 1.0)
    out[...] = (pA[...] + pB[...]) * scale + r[...]

  return pl.pallas_call(
      body,
      grid=(GRID,),
      in_specs=[_row_spec(128)] * 4,
      out_specs=_row_spec(128),
      out_shape=jax.ShapeDtypeStruct((NPAD, 128), f32),
  )


@functools.cache
def _kernels():
  return (_make_sc_layer1(), _make_sc_mid(), _make_sc_l4(),
          _make_tc1(), _make_tc_mid(), _make_tc3(), _make_tc4())


def kernel(x, edge_index, W1l, b1l, W1r, ln_g, ln_b, Wres, bres,
           W2l, b2l, W2r, W3l, b3l, W3r, W4l, b4l, W4r):
  sc1, sc_mid, sc4, tc1, tc_mid, tc3, tc4 = _kernels()

  src = edge_index[0].astype(i32)
  dst = edge_index[1].astype(i32)
  src_p = jnp.concatenate([src, jnp.zeros((EP - E,), i32)]).reshape(
      16, CPT * CH)
  dst_p = jnp.concatenate([dst, jnp.full((EP - E,), N, i32)]).reshape(
      16, CPT, CH)
  src_p4 = jnp.concatenate([src, jnp.zeros((EP4 - E,), i32)]).reshape(
      32, CPT4 * CH)
  dst_p4 = jnp.concatenate([dst, jnp.full((EP4 - E,), N, i32)]).reshape(
      32, CPT4, CH)

  xpad = jnp.pad(x, ((0, NPAD - N), (0, 0)))
  z128 = jnp.zeros((ROWS_ACC, 128), f32)
  e0 = jnp.zeros((CH, 128), f32).at[:, 0].set(1.0)

  r1 = lambda a: a.reshape(1, -1)
  ecol = lambda w: jnp.pad(w, ((0, 0), (0, 127)))

  agg1, cnt = sc1(src_p, dst_p, xpad, z128, e0)
  h1L, h1R = tc1(xpad, agg1, cnt, W1l, r1(b1l), W1r, r1(ln_g), r1(ln_b),
                 Wres, r1(bres))
  aggL2, aggR2 = sc_mid(src_p, dst_p, h1L, h1R, z128)
  h2L, h2R = tc_mid(h1L, h1R, aggL2, aggR2, cnt,
                    W2l[:128], W2l[128:], r1(b2l), W2r[:128], W2r[128:])
  aggL3, aggR3 = sc_mid(src_p, dst_p, h2L, h2R, z128)
  y128, r128 = tc3(h2L, h2R, aggL3, aggR3, cnt,
                   W3l[:128], W3l[128:], r1(b3l), W3r[:128], W3r[128:],
                   ecol(W4l), ecol(W4r), ecol(r1(b4l)))
  pA, pB = sc4(src_p4, dst_p4, y128, z128)
  out = tc4(pA, pB, cnt, r128)
  return out[:N, 0]


# speedup vs baseline: 1.6956x; 1.6956x over previous
import functools

import jax
import jax.numpy as jnp
from jax import lax
from jax.experimental import pallas as pl
from jax.experimental.pallas import tpu as pltpu
from jax.experimental.pallas import tpu_sc as plsc

N = 10000
E = 320000
NPAD = 10240
ROWS_PT = NPAD // 16
NACC = 10112
ROWS_ACC = NACC // 16
CH = 120
KG = 3
SUPER = 8
GROUPS = (3, 3, 2)
CPT = 168
EP = 16 * CPT * CH
CPT4 = 88
EP4 = 32 * CPT4 * CH
NSUP = CPT // SUPER
NSUP4 = CPT4 // SUPER

TN = 1024
GRID = NPAD // TN

f32 = jnp.float32
i32 = jnp.int32

_MESH = dict(core_axis_name="c", subcore_axis_name="s")


def _seg_loop(tab, src_h, dst_h, w, nsup, src_v, dst_v, bufs, acc, sem, sem2,
              gather=True):
  def sup(u, carry):
    if gather:
      pltpu.sync_copy(src_h.at[w, pl.ds(u * SUPER, SUPER)], src_v)
    pltpu.sync_copy(dst_h.at[w, pl.ds(u * SUPER, SUPER)], dst_v)

    base = 0
    for gs in GROUPS:
      if gather:
        gd = [pltpu.async_copy(tab.at[src_v.at[base + k]], bufs[k], sem)
              for k in range(gs)]
      sd = []
      for k in range(gs):
        if gather:
          gd[k].wait()
        sd.append(pltpu.async_copy(bufs[k], acc.at[dst_v.at[base + k]],
                                   sem2, add=True))
      for d in sd:
        d.wait()
      base += gs
    return carry
  lax.fori_loop(0, nsup, sup, 0)


def _make_sc_layer1():
  out_type = [jax.ShapeDtypeStruct((NPAD, 128), f32),
              jax.ShapeDtypeStruct((NPAD, 128), f32)]
  scratch = (
      [pltpu.VMEM((SUPER, CH), i32), pltpu.VMEM((SUPER, CH), i32)]
      + [pltpu.VMEM((CH, 128), f32) for _ in range(KG)]
      + [pltpu.VMEM_SHARED((NACC, 128), f32),
         pltpu.SemaphoreType.DMA, pltpu.SemaphoreType.DMA])

  def body(src_h, dst_h, xtab, zrows, e0_h, agg_out, cnt_out,
           src_v, dst_v, *rest):
    bufs = list(rest[:KG])
    acc, sem, sem2 = rest[KG:]
    c = lax.axis_index("c")
    s = lax.axis_index("s")
    rows = pl.ds(s * ROWS_ACC, ROWS_ACC)

    pltpu.sync_copy(zrows, acc.at[rows])

    @pl.when(c == 1)
    def _():
      for b in bufs:
        pltpu.sync_copy(e0_h, b)

    plsc.subcore_barrier()

    @pl.when(c == 0)
    def _():
      _seg_loop(xtab, src_h, dst_h, s, NSUP, src_v, dst_v, bufs, acc,
                sem, sem2)

    @pl.when(c == 1)
    def _():
      _seg_loop(xtab, src_h, dst_h, s, NSUP, src_v, dst_v, bufs, acc,
                sem, sem2, gather=False)

    plsc.subcore_barrier()

    @pl.when(c == 0)
    def _():
      pltpu.sync_copy(acc.at[rows], agg_out.at[rows])

    @pl.when(c == 1)
    def _():
      pltpu.sync_copy(acc.at[rows], cnt_out.at[rows])

  return pl.kernel(body, out_type=out_type,
                   mesh=plsc.VectorSubcoreMesh(**_MESH),
                   scratch_types=scratch)


def _make_sc_mid():
  out_type = [jax.ShapeDtypeStruct((NPAD, 128), f32),
              jax.ShapeDtypeStruct((NPAD, 128), f32)]
  scratch = (
      [pltpu.VMEM((SUPER, CH), i32), pltpu.VMEM((SUPER, CH), i32)]
      + [pltpu.VMEM((CH, 128), f32) for _ in range(KG)]
      + [pltpu.VMEM_SHARED((NACC, 128), f32),
         pltpu.SemaphoreType.DMA, pltpu.SemaphoreType.DMA])

  def body(src_h, dst_h, tabL, tabR, zrows, outL, outR, src_v, dst_v, *rest):
    bufs = list(rest[:KG])
    acc, sem, sem2 = rest[KG:]
    c = lax.axis_index("c")
    s = lax.axis_index("s")
    rows = pl.ds(s * ROWS_ACC, ROWS_ACC)

    pltpu.sync_copy(zrows, acc.at[rows])
    plsc.subcore_barrier()

    @pl.when(c == 0)
    def _():
      _seg_loop(tabL, src_h, dst_h, s, NSUP, src_v, dst_v, bufs, acc,
                sem, sem2)

    @pl.when(c == 1)
    def _():
      _seg_loop(tabR, src_h, dst_h, s, NSUP, src_v, dst_v, bufs, acc,
                sem, sem2)

    plsc.subcore_barrier()

    @pl.when(c == 0)
    def _():
      pltpu.sync_copy(acc.at[rows], outL.at[rows])

    @pl.when(c == 1)
    def _():
      pltpu.sync_copy(acc.at[rows], outR.at[rows])

  return pl.kernel(body, out_type=out_type,
                   mesh=plsc.VectorSubcoreMesh(**_MESH),
                   scratch_types=scratch)


def _make_sc_l4():
  out_type = [jax.ShapeDtypeStruct((NPAD, 128), f32),
              jax.ShapeDtypeStruct((NPAD, 128), f32)]
  scratch = (
      [pltpu.VMEM((SUPER, CH), i32), pltpu.VMEM((SUPER, CH), i32)]
      + [pltpu.VMEM((CH, 128), f32) for _ in range(KG)]
      + [pltpu.VMEM_SHARED((NACC, 128), f32),
         pltpu.SemaphoreType.DMA, pltpu.SemaphoreType.DMA])

  def body(src_h, dst_h, ytab, zrows, pA, pB, src_v, dst_v, *rest):
    bufs = list(rest[:KG])
    acc, sem, sem2 = rest[KG:]
    c = lax.axis_index("c")
    s = lax.axis_index("s")
    w = c * 16 + s
    rows = pl.ds(s * ROWS_ACC, ROWS_ACC)

    pltpu.sync_copy(zrows, acc.at[rows])
    plsc.subcore_barrier()

    _seg_loop(ytab, src_h, dst_h, w, NSUP4, src_v, dst_v, bufs, acc,
              sem, sem2)

    plsc.subcore_barrier()

    @pl.when(c == 0)
    def _():
      pltpu.sync_copy(acc.at[rows], pA.at[rows])

    @pl.when(c == 1)
    def _():
      pltpu.sync_copy(acc.at[rows], pB.at[rows])

  return pl.kernel(body, out_type=out_type,
                   mesh=plsc.VectorSubcoreMesh(**_MESH),
                   scratch_types=scratch)


def _row_spec(w):
  return pl.BlockSpec((TN, w), lambda i: (i, 0))


def _w_spec(r, cols):
  return pl.BlockSpec((r, cols), lambda i: (0, 0))


def _make_tc1():
  def body(x_ref, agg, cnt, wl, bl, wr, g, bln, wres, brs, outL, outR):
    scale = 1.0 / jnp.maximum(cnt[...][:, :1], 1.0)
    seg = jnp.dot(agg[...], wl[...], preferred_element_type=f32)
    pre = (seg * scale + bl[...]
           + jnp.dot(x_ref[...], wr[...], preferred_element_type=f32))
    mu = jnp.mean(pre, axis=1, keepdims=True)
    var = jnp.mean((pre - mu) ** 2, axis=1, keepdims=True)
    h = (pre - mu) * lax.rsqrt(var + 1e-5) * g[...] + bln[...]
    h = (jnp.maximum(h, 0.0)
         + jnp.dot(x_ref[...], wres[...], preferred_element_type=f32)
         + brs[...])
    outL[...] = h[:, :128]
    outR[...] = h[:, 128:]

  return pl.pallas_call(
      body,
      grid=(GRID,),
      in_specs=[
          _row_spec(128), _row_spec(128), _row_spec(128),
          _w_spec(128, 256), _w_spec(1, 256), _w_spec(128, 256),
          _w_spec(1, 256), _w_spec(1, 256),
          _w_spec(128, 256), _w_spec(1, 256),
      ],
      out_specs=[_row_spec(128), _row_spec(128)],
      out_shape=[jax.ShapeDtypeStruct((NPAD, 128), f32)] * 2,
  )


def _make_tc_mid():
  def body(hL, hR, aL, aR, cnt, wlt, wlb, bl, wrt, wrb, outL, outR):
    scale = 1.0 / jnp.maximum(cnt[...][:, :1], 1.0)
    seg = (jnp.dot(aL[...], wlt[...], preferred_element_type=f32)
           + jnp.dot(aR[...], wlb[...], preferred_element_type=f32))
    h = (seg * scale + bl[...]
         + jnp.dot(hL[...], wrt[...], preferred_element_type=f32)
         + jnp.dot(hR[...], wrb[...], preferred_element_type=f32))
    h = jnp.maximum(h, 0.0)
    outL[...] = h[:, :128]
    outR[...] = h[:, 128:]

  return pl.pallas_call(
      body,
      grid=(GRID,),
      in_specs=[
          _row_spec(128), _row_spec(128), _row_spec(128), _row_spec(128),
          _row_spec(128),
          _w_spec(128, 256), _w_spec(128, 256), _w_spec(1, 256),
          _w_spec(128, 256), _w_spec(128, 256),
      ],
      out_specs=[_row_spec(128), _row_spec(128)],
      out_shape=[jax.ShapeDtypeStruct((NPAD, 128), f32)] * 2,
  )


def _make_tc3():
  def body(hL, hR, aL, aR, cnt, wlt, wlb, bl, wrt, wrb, w4l, w4r, b4,
           y_out, r_out):
    scale = 1.0 / jnp.maximum(cnt[...][:, :1], 1.0)
    seg = (jnp.dot(aL[...], wlt[...], preferred_element_type=f32)
           + jnp.dot(aR[...], wlb[...], preferred_element_type=f32))
    h = (seg * scale + bl[...]
         + jnp.dot(hL[...], wrt[...], preferred_element_type=f32)
         + jnp.dot(hR[...], wrb[...], preferred_element_type=f32))
    h = jnp.maximum(h, 0.0)
    y_out[...] = jnp.dot(h, w4l[...], preferred_element_type=f32)
    r_out[...] = jnp.dot(h, w4r[...], preferred_element_type=f32) + b4[...]

  return pl.pallas_call(
      body,
      grid=(GRID,),
      in_specs=[
          _row_spec(128), _row_spec(128), _row_spec(128), _row_spec(128),
          _row_spec(128),
          _w_spec(128, 256), _w_spec(128, 256), _w_spec(1, 256),
          _w_spec(128, 256), _w_spec(128, 256),
          _w_spec(256, 128), _w_spec(256, 128), _w_spec(1, 128),
      ],
      out_specs=[_row_spec(128), _row_spec(128)],
      out_shape=[jax.ShapeDtypeStruct((NPAD, 128), f32)] * 2,
  )


def _make_tc4():
  def body(pA, pB, cnt, r, out):
    scale = 1.0 / jnp.maximum(cnt[...][:, :1], 1.0)
    out[...] = (pA[...] + pB[...]) * scale + r[...]

  return pl.pallas_call(
      body,
      grid=(GRID,),
      in_specs=[_row_spec(128)] * 4,
      out_specs=_row_spec(128),
      out_shape=jax.ShapeDtypeStruct((NPAD, 128), f32),
  )


@functools.cache
def _kernels():
  return (_make_sc_layer1(), _make_sc_mid(), _make_sc_l4(),
          _make_tc1(), _make_tc_mid(), _make_tc3(), _make_tc4())


def kernel(x, edge_index, W1l, b1l, W1r, ln_g, ln_b, Wres, bres,
           W2l, b2l, W2r, W3l, b3l, W3r, W4l, b4l, W4r):
  sc1, sc_mid, sc4, tc1, tc_mid, tc3, tc4 = _kernels()

  src = edge_index[0].astype(i32)
  dst = edge_index[1].astype(i32)
  src_p = jnp.concatenate([src, jnp.zeros((EP - E,), i32)]).reshape(
      16, CPT, CH)
  dst_p = jnp.concatenate([dst, jnp.full((EP - E,), N, i32)]).reshape(
      16, CPT, CH)
  src_p4 = jnp.concatenate([src, jnp.zeros((EP4 - E,), i32)]).reshape(
      32, CPT4, CH)
  dst_p4 = jnp.concatenate([dst, jnp.full((EP4 - E,), N, i32)]).reshape(
      32, CPT4, CH)

  xpad = jnp.pad(x, ((0, NPAD - N), (0, 0)))
  z128 = jnp.zeros((ROWS_ACC, 128), f32)
  e0 = jnp.zeros((CH, 128), f32).at[:, 0].set(1.0)

  r1 = lambda a: a.reshape(1, -1)
  ecol = lambda w: jnp.pad(w, ((0, 0), (0, 127)))

  agg1, cnt = sc1(src_p, dst_p, xpad, z128, e0)
  h1L, h1R = tc1(xpad, agg1, cnt, W1l, r1(b1l), W1r, r1(ln_g), r1(ln_b),
                 Wres, r1(bres))
  aggL2, aggR2 = sc_mid(src_p, dst_p, h1L, h1R, z128)
  h2L, h2R = tc_mid(h1L, h1R, aggL2, aggR2, cnt,
                    W2l[:128], W2l[128:], r1(b2l), W2r[:128], W2r[128:])
  aggL3, aggR3 = sc_mid(src_p, dst_p, h2L, h2R, z128)
  y128, r128 = tc3(h2L, h2R, aggL3, aggR3, cnt,
                   W3l[:128], W3l[128:], r1(b3l), W3r[:128], W3r[128:],
                   ecol(W4l), ecol(W4r), ecol(r1(b4l)))
  pA, pB = sc4(src_p4, dst_p4, y128, z128)
  out = tc4(pA, pB, cnt, r128)
  return out[:N, 0]


# speedup vs baseline: 3.3969x; 2.0034x over previous
import functools

import jax
import jax.numpy as jnp
from jax import lax
from jax.experimental import pallas as pl
from jax.experimental.pallas import tpu as pltpu
from jax.experimental.pallas import tpu_sc as plsc

N = 10000
E = 320000
NPAD = 10240
ROWS_PT = NPAD // 16
NACC = 10112
ROWS_ACC = NACC // 16
CH = 120
KG = 3
SUPER = 8
GROUPS = (3, 3, 2)
CPT = 168
EP = 16 * CPT * CH
CPT4 = 88
EP4 = 32 * CPT4 * CH
NSUP = CPT // SUPER
NSUP4 = CPT4 // SUPER

TN = 1024
GRID = NPAD // TN

f32 = jnp.float32
i32 = jnp.int32

_MESH = dict(core_axis_name="c", subcore_axis_name="s")


def _seg_loop(tab, src_h, dst_h, w, nsup, src_v, dst_v, bufs, acc, sem, sem2,
              gather=True):
  def sup(u, carry):
    if gather:
      pltpu.sync_copy(src_h.at[w, pl.ds(u * SUPER, SUPER)], src_v)
    pltpu.sync_copy(dst_h.at[w, pl.ds(u * SUPER, SUPER)], dst_v)

    base = 0
    for gs in GROUPS:
      if gather:
        gd = [pltpu.async_copy(tab.at[src_v.at[base + k]], bufs[k], sem)
              for k in range(gs)]
      sd = []
      for k in range(gs):
        if gather:
          gd[k].wait()
        sd.append(pltpu.async_copy(bufs[k], acc.at[dst_v.at[base + k]],
                                   sem2, add=True))
      for d in sd:
        d.wait()
      base += gs
    return carry
  lax.fori_loop(0, nsup, sup, 0)


def _make_sc_layer1():
  out_type = [jax.ShapeDtypeStruct((NPAD, 128), f32),
              jax.ShapeDtypeStruct((NPAD, 128), f32)]
  scratch = (
      [pltpu.VMEM((SUPER, CH), i32), pltpu.VMEM((SUPER, CH), i32)]
      + [pltpu.VMEM((CH, 128), f32) for _ in range(KG)]
      + [pltpu.VMEM_SHARED((NACC, 128), f32),
         pltpu.SemaphoreType.DMA, pltpu.SemaphoreType.DMA])

  def body(src_h, dst_h, xtab, zrows, e0_h, agg_out, cnt_out,
           src_v, dst_v, *rest):
    bufs = list(rest[:KG])
    acc, sem, sem2 = rest[KG:]
    c = lax.axis_index("c")
    s = lax.axis_index("s")
    rows = pl.ds(s * ROWS_ACC, ROWS_ACC)

    pltpu.sync_copy(zrows, acc.at[rows])

    @pl.when(c == 1)
    def _():
      for b in bufs:
        pltpu.sync_copy(e0_h, b)

    plsc.subcore_barrier()

    @pl.when(c == 0)
    def _():
      _seg_loop(xtab, src_h, dst_h, s, NSUP, src_v, dst_v, bufs, acc,
                sem, sem2)

    @pl.when(c == 1)
    def _():
      _seg_loop(xtab, src_h, dst_h, s, NSUP, src_v, dst_v, bufs, acc,
                sem, sem2, gather=False)

    plsc.subcore_barrier()

    @pl.when(c == 0)
    def _():
      pltpu.sync_copy(acc.at[rows], agg_out.at[rows])

    @pl.when(c == 1)
    def _():
      pltpu.sync_copy(acc.at[rows], cnt_out.at[rows])

  return pl.kernel(body, out_type=out_type,
                   mesh=plsc.VectorSubcoreMesh(**_MESH),
                   scratch_types=scratch)


def _make_sc_mid():
  out_type = [jax.ShapeDtypeStruct((NPAD, 128), f32),
              jax.ShapeDtypeStruct((NPAD, 128), f32)]
  scratch = (
      [pltpu.VMEM((SUPER, CH), i32), pltpu.VMEM((SUPER, CH), i32)]
      + [pltpu.VMEM((CH, 128), f32) for _ in range(KG)]
      + [pltpu.VMEM_SHARED((NACC, 128), f32),
         pltpu.SemaphoreType.DMA, pltpu.SemaphoreType.DMA])

  def body(src_h, dst_h, tabL, tabR, zrows, outL, outR, src_v, dst_v, *rest):
    bufs = list(rest[:KG])
    acc, sem, sem2 = rest[KG:]
    c = lax.axis_index("c")
    s = lax.axis_index("s")
    rows = pl.ds(s * ROWS_ACC, ROWS_ACC)

    pltpu.sync_copy(zrows, acc.at[rows])
    plsc.subcore_barrier()

    @pl.when(c == 0)
    def _():
      _seg_loop(tabL, src_h, dst_h, s, NSUP, src_v, dst_v, bufs, acc,
                sem, sem2)

    @pl.when(c == 1)
    def _():
      _seg_loop(tabR, src_h, dst_h, s, NSUP, src_v, dst_v, bufs, acc,
                sem, sem2)

    plsc.subcore_barrier()

    @pl.when(c == 0)
    def _():
      pltpu.sync_copy(acc.at[rows], outL.at[rows])

    @pl.when(c == 1)
    def _():
      pltpu.sync_copy(acc.at[rows], outR.at[rows])

  return pl.kernel(body, out_type=out_type,
                   mesh=plsc.VectorSubcoreMesh(**_MESH),
                   scratch_types=scratch)


def _make_sc_l4():
  out_type = [jax.ShapeDtypeStruct((NPAD, 128), f32),
              jax.ShapeDtypeStruct((NPAD, 128), f32)]
  scratch = (
      [pltpu.VMEM((SUPER, CH), i32), pltpu.VMEM((SUPER, CH), i32)]
      + [pltpu.VMEM((CH, 128), f32) for _ in range(KG)]
      + [pltpu.VMEM_SHARED((NACC, 128), f32),
         pltpu.SemaphoreType.DMA, pltpu.SemaphoreType.DMA])

  def body(src_h, dst_h, ytab, zrows, pA, pB, src_v, dst_v, *rest):
    bufs = list(rest[:KG])
    acc, sem, sem2 = rest[KG:]
    c = lax.axis_index("c")
    s = lax.axis_index("s")
    w = c * 16 + s
    rows = pl.ds(s * ROWS_ACC, ROWS_ACC)

    pltpu.sync_copy(zrows, acc.at[rows])
    plsc.subcore_barrier()

    _seg_loop(ytab, src_h, dst_h, w, NSUP4, src_v, dst_v, bufs, acc,
              sem, sem2)

    plsc.subcore_barrier()

    @pl.when(c == 0)
    def _():
      pltpu.sync_copy(acc.at[rows], pA.at[rows])

    @pl.when(c == 1)
    def _():
      pltpu.sync_copy(acc.at[rows], pB.at[rows])

  return pl.kernel(body, out_type=out_type,
                   mesh=plsc.VectorSubcoreMesh(**_MESH),
                   scratch_types=scratch)


def _row_spec(w):
  return pl.BlockSpec((TN, w), lambda i: (i, 0))


def _w_spec(r, cols):
  return pl.BlockSpec((r, cols), lambda i: (0, 0))


def _make_tc1():
  def body(x_ref, agg, cnt, wl, bl, wr, g, bln, wres, brs, outL, outR):
    scale = 1.0 / jnp.maximum(cnt[...][:, :1], 1.0)
    seg = jnp.dot(agg[...], wl[...], preferred_element_type=f32)
    pre = (seg * scale + bl[...]
           + jnp.dot(x_ref[...], wr[...], preferred_element_type=f32))
    mu = jnp.mean(pre, axis=1, keepdims=True)
    var = jnp.mean((pre - mu) ** 2, axis=1, keepdims=True)
    h = (pre - mu) * lax.rsqrt(var + 1e-5) * g[...] + bln[...]
    h = (jnp.maximum(h, 0.0)
         + jnp.dot(x_ref[...], wres[...], preferred_element_type=f32)
         + brs[...])
    outL[...] = h[:, :128]
    outR[...] = h[:, 128:]

  return pl.pallas_call(
      body,
      grid=(GRID,),
      in_specs=[
          _row_spec(128), _row_spec(128), _row_spec(128),
          _w_spec(128, 256), _w_spec(1, 256), _w_spec(128, 256),
          _w_spec(1, 256), _w_spec(1, 256),
          _w_spec(128, 256), _w_spec(1, 256),
      ],
      out_specs=[_row_spec(128), _row_spec(128)],
      out_shape=[jax.ShapeDtypeStruct((NPAD, 128), f32)] * 2,
  )


def _make_tc_mid():
  def body(hL, hR, aL, aR, cnt, wlt, wlb, bl, wrt, wrb, outL, outR):
    scale = 1.0 / jnp.maximum(cnt[...][:, :1], 1.0)
    seg = (jnp.dot(aL[...], wlt[...], preferred_element_type=f32)
           + jnp.dot(aR[...], wlb[...], preferred_element_type=f32))
    h = (seg * scale + bl[...]
         + jnp.dot(hL[...], wrt[...], preferred_element_type=f32)
         + jnp.dot(hR[...], wrb[...], preferred_element_type=f32))
    h = jnp.maximum(h, 0.0)
    outL[...] = h[:, :128]
    outR[...] = h[:, 128:]

  return pl.pallas_call(
      body,
      grid=(GRID,),
      in_specs=[
          _row_spec(128), _row_spec(128), _row_spec(128), _row_spec(128),
          _row_spec(128),
          _w_spec(128, 256), _w_spec(128, 256), _w_spec(1, 256),
          _w_spec(128, 256), _w_spec(128, 256),
      ],
      out_specs=[_row_spec(128), _row_spec(128)],
      out_shape=[jax.ShapeDtypeStruct((NPAD, 128), f32)] * 2,
  )


def _make_tc3():
  def body(hL, hR, aL, aR, cnt, wlt, wlb, bl, wrt, wrb, w4l, w4r, b4,
           y_out, r_out):
    scale = 1.0 / jnp.maximum(cnt[...][:, :1], 1.0)
    seg = (jnp.dot(aL[...], wlt[...], preferred_element_type=f32)
           + jnp.dot(aR[...], wlb[...], preferred_element_type=f32))
    h = (seg * scale + bl[...]
         + jnp.dot(hL[...], wrt[...], preferred_element_type=f32)
         + jnp.dot(hR[...], wrb[...], preferred_element_type=f32))
    h = jnp.maximum(h, 0.0)
    y_out[...] = jnp.dot(h, w4l[...], preferred_element_type=f32)
    r_out[...] = jnp.dot(h, w4r[...], preferred_element_type=f32) + b4[...]

  return pl.pallas_call(
      body,
      grid=(GRID,),
      in_specs=[
          _row_spec(128), _row_spec(128), _row_spec(128), _row_spec(128),
          _row_spec(128),
          _w_spec(128, 256), _w_spec(128, 256), _w_spec(1, 256),
          _w_spec(128, 256), _w_spec(128, 256),
          _w_spec(256, 128), _w_spec(256, 128), _w_spec(1, 128),
      ],
      out_specs=[_row_spec(128), _row_spec(128)],
      out_shape=[jax.ShapeDtypeStruct((NPAD, 128), f32)] * 2,
  )


def _make_tc4():
  def body(pA, pB, cnt, r, out):
    scale = 1.0 / jnp.maximum(cnt[...][:, :1], 1.0)
    out[...] = (pA[...] + pB[...]) * scale + r[...]

  return pl.pallas_call(
      body,
      grid=(GRID,),
      in_specs=[_row_spec(128)] * 4,
      out_specs=_row_spec(128),
      out_shape=jax.ShapeDtypeStruct((NPAD, 128), f32),
  )


@functools.cache
def _kernels():
  return (_make_sc_layer1(), _make_sc_mid(), _make_sc_l4(),
          _make_tc1(), _make_tc_mid(), _make_tc3(), _make_tc4())


def kernel(x, edge_index, W1l, b1l, W1r, ln_g, ln_b, Wres, bres,
           W2l, b2l, W2r, W3l, b3l, W3r, W4l, b4l, W4r):
  sc1, sc_mid, sc4, tc1, tc_mid, tc3, tc4 = _kernels()

  src = edge_index[0].astype(i32)
  dst = edge_index[1].astype(i32)

  def padded(v, ep, pad_junk):
    pidx = jnp.arange(ep - E, dtype=i32)
    fill = (pidx * 509) % (NACC - N) + N if pad_junk else (pidx * 509) % N
    return jnp.concatenate([v, fill])

  src_p = padded(src, EP, False).reshape(16, CPT, CH)
  dst_p = padded(dst, EP, True).reshape(16, CPT, CH)
  src_p4 = padded(src, EP4, False).reshape(32, CPT4, CH)
  dst_p4 = padded(dst, EP4, True).reshape(32, CPT4, CH)

  xpad = jnp.pad(x, ((0, NPAD - N), (0, 0)))
  z128 = jnp.zeros((ROWS_ACC, 128), f32)
  e0 = jnp.zeros((CH, 128), f32).at[:, 0].set(1.0)

  r1 = lambda a: a.reshape(1, -1)
  ecol = lambda w: jnp.pad(w, ((0, 0), (0, 127)))

  agg1, cnt = sc1(src_p, dst_p, xpad, z128, e0)
  h1L, h1R = tc1(xpad, agg1, cnt, W1l, r1(b1l), W1r, r1(ln_g), r1(ln_b),
                 Wres, r1(bres))
  aggL2, aggR2 = sc_mid(src_p, dst_p, h1L, h1R, z128)
  h2L, h2R = tc_mid(h1L, h1R, aggL2, aggR2, cnt,
                    W2l[:128], W2l[128:], r1(b2l), W2r[:128], W2r[128:])
  aggL3, aggR3 = sc_mid(src_p, dst_p, h2L, h2R, z128)
  y128, r128 = tc3(h2L, h2R, aggL3, aggR3, cnt,
                   W3l[:128], W3l[128:], r1(b3l), W3r[:128], W3r[128:],
                   ecol(W4l), ecol(W4r), ecol(r1(b4l)))
  pA, pB = sc4(src_p4, dst_p4, y128, z128)
  out = tc4(pA, pB, cnt, r128)
  return out[:N, 0]


# speedup vs baseline: 3.4161x; 1.0057x over previous
import functools

import jax
import jax.numpy as jnp
from jax import lax
from jax.experimental import pallas as pl
from jax.experimental.pallas import tpu as pltpu
from jax.experimental.pallas import tpu_sc as plsc

N = 10000
E = 320000
NPAD = 10240
ROWS_PT = NPAD // 16
NACC = 10112
ROWS_ACC = NACC // 16
CH = 88
KG = 4
SUPER = 8
GROUPS = (4, 4)
CPT = 232
EP = 16 * CPT * CH
CPT4 = 120
EP4 = 32 * CPT4 * CH
NSUP = CPT // SUPER
NSUP4 = CPT4 // SUPER

TN = 1024
GRID = NPAD // TN

f32 = jnp.float32
i32 = jnp.int32

_MESH = dict(core_axis_name="c", subcore_axis_name="s")


def _seg_loop(tab, src_h, dst_h, w, nsup, src_v, dst_v, bufs, acc, sem, sem2,
              gather=True):
  def sup(u, carry):
    if gather:
      pltpu.sync_copy(src_h.at[w, pl.ds(u * SUPER, SUPER)], src_v)
    pltpu.sync_copy(dst_h.at[w, pl.ds(u * SUPER, SUPER)], dst_v)

    base = 0
    for gs in GROUPS:
      if gather:
        gd = [pltpu.async_copy(tab.at[src_v.at[base + k]], bufs[k], sem)
              for k in range(gs)]
      sd = []
      for k in range(gs):
        if gather:
          gd[k].wait()
        sd.append(pltpu.async_copy(bufs[k], acc.at[dst_v.at[base + k]],
                                   sem2, add=True))
      for d in sd:
        d.wait()
      base += gs
    return carry
  lax.fori_loop(0, nsup, sup, 0)


def _make_sc_layer1():
  out_type = [jax.ShapeDtypeStruct((NPAD, 128), f32),
              jax.ShapeDtypeStruct((NPAD, 128), f32)]
  scratch = (
      [pltpu.VMEM((SUPER, CH), i32), pltpu.VMEM((SUPER, CH), i32)]
      + [pltpu.VMEM((CH, 128), f32) for _ in range(KG)]
      + [pltpu.VMEM_SHARED((NACC, 128), f32),
         pltpu.SemaphoreType.DMA, pltpu.SemaphoreType.DMA])

  def body(src_h, dst_h, xtab, zrows, e0_h, agg_out, cnt_out,
           src_v, dst_v, *rest):
    bufs = list(rest[:KG])
    acc, sem, sem2 = rest[KG:]
    c = lax.axis_index("c")
    s = lax.axis_index("s")
    rows = pl.ds(s * ROWS_ACC, ROWS_ACC)

    pltpu.sync_copy(zrows, acc.at[rows])

    @pl.when(c == 1)
    def _():
      for b in bufs:
        pltpu.sync_copy(e0_h, b)

    plsc.subcore_barrier()

    @pl.when(c == 0)
    def _():
      _seg_loop(xtab, src_h, dst_h, s, NSUP, src_v, dst_v, bufs, acc,
                sem, sem2)

    @pl.when(c == 1)
    def _():
      _seg_loop(xtab, src_h, dst_h, s, NSUP, src_v, dst_v, bufs, acc,
                sem, sem2, gather=False)

    plsc.subcore_barrier()

    @pl.when(c == 0)
    def _():
      pltpu.sync_copy(acc.at[rows], agg_out.at[rows])

    @pl.when(c == 1)
    def _():
      pltpu.sync_copy(acc.at[rows], cnt_out.at[rows])

  return pl.kernel(body, out_type=out_type,
                   mesh=plsc.VectorSubcoreMesh(**_MESH),
                   scratch_types=scratch)


def _make_sc_mid():
  out_type = [jax.ShapeDtypeStruct((NPAD, 128), f32),
              jax.ShapeDtypeStruct((NPAD, 128), f32)]
  scratch = (
      [pltpu.VMEM((SUPER, CH), i32), pltpu.VMEM((SUPER, CH), i32)]
      + [pltpu.VMEM((CH, 128), f32) for _ in range(KG)]
      + [pltpu.VMEM_SHARED((NACC, 128), f32),
         pltpu.SemaphoreType.DMA, pltpu.SemaphoreType.DMA])

  def body(src_h, dst_h, tabL, tabR, zrows, outL, outR, src_v, dst_v, *rest):
    bufs = list(rest[:KG])
    acc, sem, sem2 = rest[KG:]
    c = lax.axis_index("c")
    s = lax.axis_index("s")
    rows = pl.ds(s * ROWS_ACC, ROWS_ACC)

    pltpu.sync_copy(zrows, acc.at[rows])
    plsc.subcore_barrier()

    @pl.when(c == 0)
    def _():
      _seg_loop(tabL, src_h, dst_h, s, NSUP, src_v, dst_v, bufs, acc,
                sem, sem2)

    @pl.when(c == 1)
    def _():
      _seg_loop(tabR, src_h, dst_h, s, NSUP, src_v, dst_v, bufs, acc,
                sem, sem2)

    plsc.subcore_barrier()

    @pl.when(c == 0)
    def _():
      pltpu.sync_copy(acc.at[rows], outL.at[rows])

    @pl.when(c == 1)
    def _():
      pltpu.sync_copy(acc.at[rows], outR.at[rows])

  return pl.kernel(body, out_type=out_type,
                   mesh=plsc.VectorSubcoreMesh(**_MESH),
                   scratch_types=scratch)


def _make_sc_l4():
  out_type = [jax.ShapeDtypeStruct((NPAD, 128), f32),
              jax.ShapeDtypeStruct((NPAD, 128), f32)]
  scratch = (
      [pltpu.VMEM((SUPER, CH), i32), pltpu.VMEM((SUPER, CH), i32)]
      + [pltpu.VMEM((CH, 128), f32) for _ in range(KG)]
      + [pltpu.VMEM_SHARED((NACC, 128), f32),
         pltpu.SemaphoreType.DMA, pltpu.SemaphoreType.DMA])

  def body(src_h, dst_h, ytab, zrows, pA, pB, src_v, dst_v, *rest):
    bufs = list(rest[:KG])
    acc, sem, sem2 = rest[KG:]
    c = lax.axis_index("c")
    s = lax.axis_index("s")
    w = c * 16 + s
    rows = pl.ds(s * ROWS_ACC, ROWS_ACC)

    pltpu.sync_copy(zrows, acc.at[rows])
    plsc.subcore_barrier()

    _seg_loop(ytab, src_h, dst_h, w, NSUP4, src_v, dst_v, bufs, acc,
              sem, sem2)

    plsc.subcore_barrier()

    @pl.when(c == 0)
    def _():
      pltpu.sync_copy(acc.at[rows], pA.at[rows])

    @pl.when(c == 1)
    def _():
      pltpu.sync_copy(acc.at[rows], pB.at[rows])

  return pl.kernel(body, out_type=out_type,
                   mesh=plsc.VectorSubcoreMesh(**_MESH),
                   scratch_types=scratch)


def _row_spec(w):
  return pl.BlockSpec((TN, w), lambda i: (i, 0))


def _w_spec(r, cols):
  return pl.BlockSpec((r, cols), lambda i: (0, 0))


def _make_tc1():
  def body(x_ref, agg, cnt, wl, bl, wr, g, bln, wres, brs, outL, outR):
    scale = 1.0 / jnp.maximum(cnt[...][:, :1], 1.0)
    seg = jnp.dot(agg[...], wl[...], preferred_element_type=f32)
    pre = (seg * scale + bl[...]
           + jnp.dot(x_ref[...], wr[...], preferred_element_type=f32))
    mu = jnp.mean(pre, axis=1, keepdims=True)
    var = jnp.mean((pre - mu) ** 2, axis=1, keepdims=True)
    h = (pre - mu) * lax.rsqrt(var + 1e-5) * g[...] + bln[...]
    h = (jnp.maximum(h, 0.0)
         + jnp.dot(x_ref[...], wres[...], preferred_element_type=f32)
         + brs[...])
    outL[...] = h[:, :128]
    outR[...] = h[:, 128:]

  return pl.pallas_call(
      body,
      grid=(GRID,),
      in_specs=[
          _row_spec(128), _row_spec(128), _row_spec(128),
          _w_spec(128, 256), _w_spec(1, 256), _w_spec(128, 256),
          _w_spec(1, 256), _w_spec(1, 256),
          _w_spec(128, 256), _w_spec(1, 256),
      ],
      out_specs=[_row_spec(128), _row_spec(128)],
      out_shape=[jax.ShapeDtypeStruct((NPAD, 128), f32)] * 2,
  )


def _make_tc_mid():
  def body(hL, hR, aL, aR, cnt, wlt, wlb, bl, wrt, wrb, outL, outR):
    scale = 1.0 / jnp.maximum(cnt[...][:, :1], 1.0)
    seg = (jnp.dot(aL[...], wlt[...], preferred_element_type=f32)
           + jnp.dot(aR[...], wlb[...], preferred_element_type=f32))
    h = (seg * scale + bl[...]
         + jnp.dot(hL[...], wrt[...], preferred_element_type=f32)
         + jnp.dot(hR[...], wrb[...], preferred_element_type=f32))
    h = jnp.maximum(h, 0.0)
    outL[...] = h[:, :128]
    outR[...] = h[:, 128:]

  return pl.pallas_call(
      body,
      grid=(GRID,),
      in_specs=[
          _row_spec(128), _row_spec(128), _row_spec(128), _row_spec(128),
          _row_spec(128),
          _w_spec(128, 256), _w_spec(128, 256), _w_spec(1, 256),
          _w_spec(128, 256), _w_spec(128, 256),
      ],
      out_specs=[_row_spec(128), _row_spec(128)],
      out_shape=[jax.ShapeDtypeStruct((NPAD, 128), f32)] * 2,
  )


def _make_tc3():
  def body(hL, hR, aL, aR, cnt, wlt, wlb, bl, wrt, wrb, w4l, w4r, b4,
           y_out, r_out):
    scale = 1.0 / jnp.maximum(cnt[...][:, :1], 1.0)
    seg = (jnp.dot(aL[...], wlt[...], preferred_element_type=f32)
           + jnp.dot(aR[...], wlb[...], preferred_element_type=f32))
    h = (seg * scale + bl[...]
         + jnp.dot(hL[...], wrt[...], preferred_element_type=f32)
         + jnp.dot(hR[...], wrb[...], preferred_element_type=f32))
    h = jnp.maximum(h, 0.0)
    y_out[...] = jnp.dot(h, w4l[...], preferred_element_type=f32)
    r_out[...] = jnp.dot(h, w4r[...], preferred_element_type=f32) + b4[...]

  return pl.pallas_call(
      body,
      grid=(GRID,),
      in_specs=[
          _row_spec(128), _row_spec(128), _row_spec(128), _row_spec(128),
          _row_spec(128),
          _w_spec(128, 256), _w_spec(128, 256), _w_spec(1, 256),
          _w_spec(128, 256), _w_spec(128, 256),
          _w_spec(256, 128), _w_spec(256, 128), _w_spec(1, 128),
      ],
      out_specs=[_row_spec(128), _row_spec(128)],
      out_shape=[jax.ShapeDtypeStruct((NPAD, 128), f32)] * 2,
  )


def _make_tc4():
  def body(pA, pB, cnt, r, out):
    scale = 1.0 / jnp.maximum(cnt[...][:, :1], 1.0)
    out[...] = (pA[...] + pB[...]) * scale + r[...]

  return pl.pallas_call(
      body,
      grid=(GRID,),
      in_specs=[_row_spec(128)] * 4,
      out_specs=_row_spec(128),
      out_shape=jax.ShapeDtypeStruct((NPAD, 128), f32),
  )


@functools.cache
def _kernels():
  return (_make_sc_layer1(), _make_sc_mid(), _make_sc_l4(),
          _make_tc1(), _make_tc_mid(), _make_tc3(), _make_tc4())


def kernel(x, edge_index, W1l, b1l, W1r, ln_g, ln_b, Wres, bres,
           W2l, b2l, W2r, W3l, b3l, W3r, W4l, b4l, W4r):
  sc1, sc_mid, sc4, tc1, tc_mid, tc3, tc4 = _kernels()

  src = edge_index[0].astype(i32)
  dst = edge_index[1].astype(i32)

  def padded(v, ep, pad_junk):
    pidx = jnp.arange(ep - E, dtype=i32)
    fill = (pidx * 509) % (NACC - N) + N if pad_junk else (pidx * 509) % N
    return jnp.concatenate([v, fill])

  src_p = padded(src, EP, False).reshape(16, CPT, CH)
  dst_p = padded(dst, EP, True).reshape(16, CPT, CH)
  src_p4 = padded(src, EP4, False).reshape(32, CPT4, CH)
  dst_p4 = padded(dst, EP4, True).reshape(32, CPT4, CH)

  xpad = jnp.pad(x, ((0, NPAD - N), (0, 0)))
  z128 = jnp.zeros((ROWS_ACC, 128), f32)
  e0 = jnp.zeros((CH, 128), f32).at[:, 0].set(1.0)

  r1 = lambda a: a.reshape(1, -1)
  ecol = lambda w: jnp.pad(w, ((0, 0), (0, 127)))

  agg1, cnt = sc1(src_p, dst_p, xpad, z128, e0)
  h1L, h1R = tc1(xpad, agg1, cnt, W1l, r1(b1l), W1r, r1(ln_g), r1(ln_b),
                 Wres, r1(bres))
  aggL2, aggR2 = sc_mid(src_p, dst_p, h1L, h1R, z128)
  h2L, h2R = tc_mid(h1L, h1R, aggL2, aggR2, cnt,
                    W2l[:128], W2l[128:], r1(b2l), W2r[:128], W2r[128:])
  aggL3, aggR3 = sc_mid(src_p, dst_p, h2L, h2R, z128)
  y128, r128 = tc3(h2L, h2R, aggL3, aggR3, cnt,
                   W3l[:128], W3l[128:], r1(b3l), W3r[:128], W3r[128:],
                   ecol(W4l), ecol(W4r), ecol(r1(b4l)))
  pA, pB = sc4(src_p4, dst_p4, y128, z128)
  out = tc4(pA, pB, cnt, r128)
  return out[:N, 0]
